# baseline (speedup 1.0000x reference)
import functools

import jax
import jax.numpy as jnp
from jax import lax
from jax.experimental import pallas as pl
from jax.experimental.pallas import tpu as pltpu
from jax.experimental.pallas import tpu_sc as plsc

_N = 20000
_PAD = 20480
_NC1 = _PAD // 16
_NC2 = _NC1 // 16
_MOUT = 200
_SLOTC = 13
_NEG = -jnp.inf


def _iota16():
    return jax.lax.broadcasted_iota(jnp.int32, (16,), 0)


def _extract_f(chunk, lane):
    sel = (_iota16() == lane).astype(jnp.float32)
    return jnp.sum(chunk * sel)


def _first_lane(mask):
    off = jnp.logical_not(mask).astype(jnp.int32) * 99
    return jnp.min(_iota16() + off)


def _sc_body(x1h, y1h, x2h, y2h, sh, thrh, selh, numh,
             x1, y1, x2, y2, ws, l1, l2, thrv, selv, numv,
             sx1, sy1, sx2, sy2, sa):
    wid = lax.axis_index("s") * 2 + lax.axis_index("c")

    @pl.when(wid == 0)
    def _work():
        pltpu.sync_copy(x1h, x1)
        pltpu.sync_copy(y1h, y1)
        pltpu.sync_copy(x2h, x2)
        pltpu.sync_copy(y2h, y2)
        pltpu.sync_copy(sh, ws)
        pltpu.sync_copy(thrh, thrv)
        it = _iota16()
        thrc = thrv[...]
        iou_thr = jnp.sum(thrc * (it == 0).astype(jnp.float32))
        score_thr = jnp.sum(thrc * (it == 1).astype(jnp.float32))

        zi = jnp.zeros((16,), jnp.int32)
        for k in range(_SLOTC + 3):
            selv[pl.ds(k * 16, 16)] = zi

        def initb(j, _):
            acc = jnp.full((16,), _NEG, jnp.float32)
            for k in range(16):
                cs = pl.ds(j * 256 + k * 16, 16)
                w = ws[cs]
                w = jnp.where(w > score_thr, w, _NEG)
                ws[cs] = w
                mk = jnp.max(w)
                acc = jnp.where(it == k, mk, acc)
            l1[pl.ds(j * 16, 16)] = acc
            return 0

        lax.fori_loop(0, _NC2, initb, 0)

        def initc(c, _):
            acc = jnp.full((16,), _NEG, jnp.float32)
            for k in range(16):
                mk = jnp.max(l1[pl.ds(c * 256 + k * 16, 16)])
                acc = jnp.where(it == k, mk, acc)
            l2[pl.ds(c * 16, 16)] = acc
            return 0

        lax.fori_loop(0, _NC2 // 16, initc, 0)

        def cond(carry):
            num, stop = carry
            return jnp.logical_and(num < _MOUT, jnp.logical_not(stop))

        def body(carry):
            num, stop = carry
            m = jnp.float32(_NEG)
            c_best = jnp.int32(0)
            for c in range(_NC2 // 16):
                mc = jnp.max(l2[pl.ds(c * 16, 16)])
                take = mc > m
                c_best = jnp.where(take, jnp.int32(c), c_best)
                m = jnp.maximum(mc, m)
            valid = m > _NEG
            c16 = pl.multiple_of(c_best * 16, 8)
            l2c = l2[pl.ds(c16, 16)]
            lane2 = _first_lane(l2c == m)
            j2 = c_best * 16 + lane2
            j216 = pl.multiple_of(j2 * 16, 8)
            l1c = l1[pl.ds(j216, 16)]
            lane1 = _first_lane(l1c == m)
            j1 = j2 * 16 + lane1
            j116 = pl.multiple_of(j1 * 16, 8)
            wchunk = ws[pl.ds(j116, 16)]
            lane0 = _first_lane(wchunk == m)
            idx = j1 * 16 + lane0

            b0 = _extract_f(x1[pl.ds(j116, 16)], lane0)
            b1 = _extract_f(y1[pl.ds(j116, 16)], lane0)
            b2 = _extract_f(x2[pl.ds(j116, 16)], lane0)
            b3 = _extract_f(y2[pl.ds(j116, 16)], lane0)
            a = (b2 - b0) * (b3 - b1)

            def hitb(k, hitacc):
                cs = pl.ds(k * 16, 16)
                xx1 = jnp.maximum(b0, sx1[cs])
                yy1 = jnp.maximum(b1, sy1[cs])
                xx2 = jnp.minimum(b2, sx2[cs])
                yy2 = jnp.minimum(b3, sy2[cs])
                inter = (jnp.clip(xx2 - xx1, 0.0, None)
                         * jnp.clip(yy2 - yy1, 0.0, None))
                union = jnp.maximum(a + sa[cs] - inter, 1e-6)
                iou = inter / union
                hit = (iou >= iou_thr) & ((k * 16 + it) < num)
                return jnp.maximum(hitacc,
                                   jnp.where(hit, 1.0, jnp.float32(0.0)))

            nslot = (num + 15) // 16
            hitacc = lax.fori_loop(0, nslot, hitb,
                                   jnp.zeros((16,), jnp.float32))
            hitcnt = jnp.max(hitacc)
            accepted = valid & (hitcnt == 0.0) & (num < _MOUT)

            @pl.when(accepted)
            def _store_slot():
                ks = pl.ds(pl.multiple_of((num // 16) * 16, 8), 16)
                lm = it == (num - (num // 16) * 16)
                selv[ks] = jnp.where(lm, idx, selv[ks])
                sx1[ks] = jnp.where(lm, b0, sx1[ks])
                sy1[ks] = jnp.where(lm, b1, sy1[ks])
                sx2[ks] = jnp.where(lm, b2, sx2[ks])
                sy2[ks] = jnp.where(lm, b3, sy2[ks])
                sa[ks] = jnp.where(lm, a, sa[ks])

            @pl.when(valid)
            def _remove():
                w2 = jnp.where(it == lane0, _NEG, wchunk)
                ws[pl.ds(j116, 16)] = w2
                nm1 = jnp.max(w2)
                l1c2 = jnp.where(it == lane1, nm1, l1c)
                l1[pl.ds(j216, 16)] = l1c2
                nm2 = jnp.max(l1c2)
                l2[pl.ds(c16, 16)] = jnp.where(it == lane2, nm2, l2c)

            num = num + accepted.astype(jnp.int32)
            stop = jnp.logical_not(valid)
            return (num, stop)

        num, _ = lax.while_loop(cond, body, (jnp.int32(0), jnp.bool_(False)))
        numv[...] = jnp.where(it == 0, num, 0)
        pltpu.sync_copy(selv, selh)
        pltpu.sync_copy(numv, numh)


def kernel(boxes, scores, max_output_size, iou_threshold, scores_threshold):
    boxes = boxes.astype(jnp.float32)
    scores = scores.astype(jnp.float32)
    n = boxes.shape[0]
    pad = _PAD - n
    bx = jnp.pad(boxes, ((0, pad), (0, 0)))
    s = jnp.pad(scores, (0, pad), constant_values=-jnp.inf)
    thr = jnp.zeros((16,), jnp.float32)
    thr = thr.at[0].set(jnp.asarray(iou_threshold, jnp.float32))
    thr = thr.at[1].set(jnp.asarray(scores_threshold, jnp.float32))

    mesh = plsc.VectorSubcoreMesh(core_axis_name="c", subcore_axis_name="s")
    f = functools.partial(
        pl.kernel, mesh=mesh,
        compiler_params=pltpu.CompilerParams(needs_layout_passes=False),
        out_type=[
            jax.ShapeDtypeStruct(((_SLOTC + 3) * 16,), jnp.int32),
            jax.ShapeDtypeStruct((16,), jnp.int32),
        ],
        scratch_types=[
            pltpu.VMEM((_PAD,), jnp.float32),
            pltpu.VMEM((_PAD,), jnp.float32),
            pltpu.VMEM((_PAD,), jnp.float32),
            pltpu.VMEM((_PAD,), jnp.float32),
            pltpu.VMEM((_PAD,), jnp.float32),
            pltpu.VMEM((_NC1,), jnp.float32),
            pltpu.VMEM((_NC2,), jnp.float32),
            pltpu.VMEM((16,), jnp.float32),
            pltpu.VMEM(((_SLOTC + 3) * 16,), jnp.int32),
            pltpu.VMEM((16,), jnp.int32),
            pltpu.VMEM((_SLOTC * 16,), jnp.float32),
            pltpu.VMEM((_SLOTC * 16,), jnp.float32),
            pltpu.VMEM((_SLOTC * 16,), jnp.float32),
            pltpu.VMEM((_SLOTC * 16,), jnp.float32),
            pltpu.VMEM((_SLOTC * 16,), jnp.float32),
        ],
    )(_sc_body)
    sel_m, num_m = f(bx[:, 0], bx[:, 1], bx[:, 2], bx[:, 3], s, thr)

    sel = sel_m[:_MOUT]
    num = jnp.minimum(num_m[0], jnp.asarray(max_output_size, jnp.int32))
    return (sel, num)


# speedup vs baseline: 1.0131x; 1.0131x over previous
import functools

import jax
import jax.numpy as jnp
from jax import lax
from jax.experimental import pallas as pl
from jax.experimental.pallas import tpu as pltpu
from jax.experimental.pallas import tpu_sc as plsc

_N = 20000
_PAD = 20480
_NC1 = _PAD // 16
_NC2 = _NC1 // 16
_MOUT = 200
_SLOTC = 13
_NEG = -jnp.inf


def _iota16():
    return jax.lax.broadcasted_iota(jnp.int32, (16,), 0)


def _extract_f(chunk, lane):
    sel = (_iota16() == lane).astype(jnp.float32)
    return jnp.sum(chunk * sel)


def _first_lane(mask):
    off = jnp.logical_not(mask).astype(jnp.int32) * 99
    return jnp.min(_iota16() + off)


def _sc_body(x1h, y1h, x2h, y2h, sh, thrh, selh, numh,
             x1, y1, x2, y2, ws, l1, l2, thrv, selv, numv,
             sx1, sy1, sx2, sy2, sa):
    wid = lax.axis_index("s") * 2 + lax.axis_index("c")

    @pl.when(wid == 0)
    def _work():
        pltpu.sync_copy(x1h, x1)
        pltpu.sync_copy(y1h, y1)
        pltpu.sync_copy(x2h, x2)
        pltpu.sync_copy(y2h, y2)
        pltpu.sync_copy(sh, ws)
        pltpu.sync_copy(thrh, thrv)
        it = _iota16()
        thrc = thrv[...]
        iou_thr = jnp.sum(thrc * (it == 0).astype(jnp.float32))
        score_thr = jnp.sum(thrc * (it == 1).astype(jnp.float32))

        zi = jnp.zeros((16,), jnp.int32)
        for k in range(_SLOTC + 3):
            selv[pl.ds(k * 16, 16)] = zi

        def initb(j, _):
            acc = jnp.full((16,), _NEG, jnp.float32)
            for k in range(16):
                cs = pl.ds(j * 256 + k * 16, 16)
                w = ws[cs]
                w = jnp.where(w > score_thr, w, _NEG)
                ws[cs] = w
                mk = jnp.max(w)
                acc = jnp.where(it == k, mk, acc)
            l1[pl.ds(j * 16, 16)] = acc
            return 0

        lax.fori_loop(0, _NC2, initb, 0)

        def initc(c, _):
            acc = jnp.full((16,), _NEG, jnp.float32)
            for k in range(16):
                mk = jnp.max(l1[pl.ds(c * 256 + k * 16, 16)])
                acc = jnp.where(it == k, mk, acc)
            l2[pl.ds(c * 16, 16)] = acc
            return 0

        lax.fori_loop(0, _NC2 // 16, initc, 0)

        def cond(carry):
            num, stop = carry
            return jnp.logical_and(num < _MOUT, jnp.logical_not(stop))

        def body(carry):
            num, stop = carry
            m = jnp.float32(_NEG)
            c_best = jnp.int32(0)
            for c in range(_NC2 // 16):
                mc = jnp.max(l2[pl.ds(c * 16, 16)])
                take = mc > m
                c_best = jnp.where(take, jnp.int32(c), c_best)
                m = jnp.maximum(mc, m)
            valid = m > _NEG
            c16 = pl.multiple_of(c_best * 16, 8)
            l2c = l2[pl.ds(c16, 16)]
            lane2 = _first_lane(l2c == m)
            j2 = c_best * 16 + lane2
            j216 = pl.multiple_of(j2 * 16, 8)
            l1c = l1[pl.ds(j216, 16)]
            lane1 = _first_lane(l1c == m)
            j1 = j2 * 16 + lane1
            j116 = pl.multiple_of(j1 * 16, 8)
            wchunk = ws[pl.ds(j116, 16)]
            lane0 = _first_lane(wchunk == m)
            idx = j1 * 16 + lane0

            b0 = _extract_f(x1[pl.ds(j116, 16)], lane0)
            b1 = _extract_f(y1[pl.ds(j116, 16)], lane0)
            b2 = _extract_f(x2[pl.ds(j116, 16)], lane0)
            b3 = _extract_f(y2[pl.ds(j116, 16)], lane0)
            a = (b2 - b0) * (b3 - b1)

            hitacc = jnp.zeros((16,), jnp.float32)
            for k in range(_SLOTC):
                cs = pl.ds(k * 16, 16)
                xx1 = jnp.maximum(b0, sx1[cs])
                yy1 = jnp.maximum(b1, sy1[cs])
                xx2 = jnp.minimum(b2, sx2[cs])
                yy2 = jnp.minimum(b3, sy2[cs])
                inter = (jnp.clip(xx2 - xx1, 0.0, None)
                         * jnp.clip(yy2 - yy1, 0.0, None))
                union = jnp.maximum(a + sa[cs] - inter, 1e-6)
                iou = inter / union
                hit = (iou >= iou_thr) & ((k * 16 + it) < num)
                hitacc = jnp.maximum(hitacc,
                                     jnp.where(hit, 1.0, jnp.float32(0.0)))
            hitcnt = jnp.max(hitacc)
            accepted = valid & (hitcnt == 0.0) & (num < _MOUT)

            @pl.when(accepted)
            def _store_slot():
                ks = pl.ds(pl.multiple_of((num // 16) * 16, 8), 16)
                lm = it == (num - (num // 16) * 16)
                selv[ks] = jnp.where(lm, idx, selv[ks])
                sx1[ks] = jnp.where(lm, b0, sx1[ks])
                sy1[ks] = jnp.where(lm, b1, sy1[ks])
                sx2[ks] = jnp.where(lm, b2, sx2[ks])
                sy2[ks] = jnp.where(lm, b3, sy2[ks])
                sa[ks] = jnp.where(lm, a, sa[ks])

            @pl.when(valid)
            def _remove():
                w2 = jnp.where(it == lane0, _NEG, wchunk)
                ws[pl.ds(j116, 16)] = w2
                nm1 = jnp.max(w2)
                l1c2 = jnp.where(it == lane1, nm1, l1c)
                l1[pl.ds(j216, 16)] = l1c2
                nm2 = jnp.max(l1c2)
                l2[pl.ds(c16, 16)] = jnp.where(it == lane2, nm2, l2c)

            num = num + accepted.astype(jnp.int32)
            stop = jnp.logical_not(valid)
            return (num, stop)

        num, _ = lax.while_loop(cond, body, (jnp.int32(0), jnp.bool_(False)))
        numv[...] = jnp.where(it == 0, num, 0)
        pltpu.sync_copy(selv, selh)
        pltpu.sync_copy(numv, numh)


def kernel(boxes, scores, max_output_size, iou_threshold, scores_threshold):
    boxes = boxes.astype(jnp.float32)
    scores = scores.astype(jnp.float32)
    n = boxes.shape[0]
    pad = _PAD - n
    bx = jnp.pad(boxes, ((0, pad), (0, 0)))
    s = jnp.pad(scores, (0, pad), constant_values=-jnp.inf)
    thr = jnp.zeros((16,), jnp.float32)
    thr = thr.at[0].set(jnp.asarray(iou_threshold, jnp.float32))
    thr = thr.at[1].set(jnp.asarray(scores_threshold, jnp.float32))

    mesh = plsc.VectorSubcoreMesh(core_axis_name="c", subcore_axis_name="s")
    f = functools.partial(
        pl.kernel, mesh=mesh,
        compiler_params=pltpu.CompilerParams(needs_layout_passes=False),
        out_type=[
            jax.ShapeDtypeStruct(((_SLOTC + 3) * 16,), jnp.int32),
            jax.ShapeDtypeStruct((16,), jnp.int32),
        ],
        scratch_types=[
            pltpu.VMEM((_PAD,), jnp.float32),
            pltpu.VMEM((_PAD,), jnp.float32),
            pltpu.VMEM((_PAD,), jnp.float32),
            pltpu.VMEM((_PAD,), jnp.float32),
            pltpu.VMEM((_PAD,), jnp.float32),
            pltpu.VMEM((_NC1,), jnp.float32),
            pltpu.VMEM((_NC2,), jnp.float32),
            pltpu.VMEM((16,), jnp.float32),
            pltpu.VMEM(((_SLOTC + 3) * 16,), jnp.int32),
            pltpu.VMEM((16,), jnp.int32),
            pltpu.VMEM((_SLOTC * 16,), jnp.float32),
            pltpu.VMEM((_SLOTC * 16,), jnp.float32),
            pltpu.VMEM((_SLOTC * 16,), jnp.float32),
            pltpu.VMEM((_SLOTC * 16,), jnp.float32),
            pltpu.VMEM((_SLOTC * 16,), jnp.float32),
        ],
    )(_sc_body)
    sel_m, num_m = f(bx[:, 0], bx[:, 1], bx[:, 2], bx[:, 3], s, thr)

    sel = sel_m[:_MOUT]
    num = jnp.minimum(num_m[0], jnp.asarray(max_output_size, jnp.int32))
    return (sel, num)
